# Initial kernel scaffold; baseline (speedup 1.0000x reference)
import jax
import jax.numpy as jnp
from jax.experimental import pallas as pl


def kernel(x, edge_index, W_src, W_dst, att_src, att_dst, bias, W_lin, b_lin):
    raise NotImplementedError("write your pallas kernel here")


# speedup vs baseline: 8.4538x; 8.4538x over previous
import functools

import jax
import jax.numpy as jnp
from jax import lax
from jax.experimental import pallas as pl
from jax.experimental.pallas import tpu as pltpu
from jax.experimental.pallas import tpu_sc as plsc

_N = 10000
_NP = 10240
_D = 128
_E = 320000
_NT = 4
_ET = 12
_L = 2
_SRC_T = (0, 0, 0, 1, 1, 1, 2, 2, 2, 3, 3, 3)
_DST_T = (1, 2, 3, 0, 2, 3, 0, 1, 3, 0, 1, 2)
_ETS_FOR_DST = tuple(tuple(et for et in range(_ET) if _DST_T[et] == nt)
                     for nt in range(_NT))

_NC = 2
_NS = 16
_NW = _NC * _NS

_E_ROW = _E // _NW
_CH = 80
_NCH_ROW = _E_ROW // _CH
_E_DEN = _E // _NS
_NCH_DEN = _E_DEN // _CH

_ROWS_PER_SUB = _NP // _NS


def _leaky_exp(t):
    return jnp.exp(jnp.where(t > 0, t, 0.2 * t))


_RB = 1024
_NB = _NP // _RB


def _proj_body(xsrc, xdst, ws, wd, asrc, adst, xs_out, as_out, ad_out):
    xs = jnp.dot(xsrc[0], ws[0], preferred_element_type=jnp.float32)
    xs_out[0] = xs
    as_out[0] = lax.dot_general(asrc[0], xs, (((1,), (1,)), ((), ())),
                                preferred_element_type=jnp.float32)
    w_eff = lax.dot_general(adst[0], wd[0], (((1,), (1,)), ((), ())),
                            preferred_element_type=jnp.float32)
    ad_out[0] = lax.dot_general(w_eff, xdst[0], (((1,), (1,)), ((), ())),
                                preferred_element_type=jnp.float32)


def _src_of(et):
    return et // 3


def _dst_of(et):
    s = et // 3
    r = et % 3
    return jnp.where(r < s, r, r + 1)


def _et_of(nt, k):
    s = jnp.where(k < nt, k, k + 1)
    r = jnp.where(nt < s, nt, nt - 1)
    return 3 * s + r


def _projections(xp, Ws, Wd, att_s, att_d):
    return pl.pallas_call(
        _proj_body,
        grid=(_ET, _NB),
        in_specs=[
            pl.BlockSpec((1, _RB, _D), lambda et, i: (_src_of(et), i, 0)),
            pl.BlockSpec((1, _RB, _D), lambda et, i: (_dst_of(et), i, 0)),
            pl.BlockSpec((1, _D, _D), lambda et, i: (et, 0, 0)),
            pl.BlockSpec((1, _D, _D), lambda et, i: (et, 0, 0)),
            pl.BlockSpec((1, 1, _D), lambda et, i: (et, 0, 0)),
            pl.BlockSpec((1, 1, _D), lambda et, i: (et, 0, 0)),
        ],
        out_specs=[
            pl.BlockSpec((1, _RB, _D), lambda et, i: (et, i, 0)),
            pl.BlockSpec((1, 1, _RB), lambda et, i: (et, 0, i)),
            pl.BlockSpec((1, 1, _RB), lambda et, i: (et, 0, i)),
        ],
        out_shape=[
            jax.ShapeDtypeStruct((_ET, _NP, _D), jnp.float32),
            jax.ShapeDtypeStruct((_ET, 1, _NP), jnp.float32),
            jax.ShapeDtypeStruct((_ET, 1, _NP), jnp.float32),
        ],
    )(xp, xp, Ws, Wd, att_s[:, None, :], att_d[:, None, :])


def _sc_body(src_hbm, dst_hbm, as_hbm, ad_hbm, xs_hbm, raw_hbm,
             as_v, ad_v, den_v, eib, ee_v, rows_v,
             den_sp, out_sp, sem):
    c = lax.axis_index("c")
    s = lax.axis_index("s")
    tile = c * _NS + s
    z16 = jnp.zeros((16,), jnp.float32)
    zi16 = jnp.zeros((16,), jnp.int32)

    def per_et(et, _):
        def zrow_init(i, _):
            for q in range(_D // 16):
                rows_v[i, pl.ds(q * 16, 16)] = z16
            return 0
        lax.fori_loop(0, _CH, zrow_init, 0)

        def zden_init(i, _):
            den_v[pl.ds(i * 16, 16)] = z16
            return 0
        lax.fori_loop(0, _ROWS_PER_SUB // 16, zden_init, 0)

        for r in range(_ROWS_PER_SUB // _CH):
            pltpu.sync_copy(rows_v,
                            out_sp.at[pl.ds(s * _ROWS_PER_SUB + r * _CH, _CH)])
        pltpu.sync_copy(den_v.at[pl.ds(0, _ROWS_PER_SUB)],
                        den_sp.at[pl.ds(s * _ROWS_PER_SUB, _ROWS_PER_SUB)])
        pltpu.sync_copy(as_hbm.at[pl.ds(et * _NP, _NP)], as_v)
        pltpu.sync_copy(ad_hbm.at[pl.ds(et * _NP, _NP)], ad_v)
        plsc.subcore_barrier()

        base_den = et * _E + s * _E_DEN

        def den_chunk(j, _):
            off = base_den + j * _CH
            pltpu.sync_copy(src_hbm.at[pl.ds(off, _CH)], eib.at[0])
            pltpu.sync_copy(dst_hbm.at[pl.ds(off, _CH)], eib.at[1])
            for q in range(_CH // 16):
                si = eib[0, pl.ds(q * 16, 16)]
                di = eib[1, pl.ds(q * 16, 16)]
                av = plsc.load_gather(as_v, [si])
                bv = plsc.load_gather(ad_v, [di])
                ee_v[0, pl.ds(q * 16, 16)] = _leaky_exp(av + bv)
            pltpu.sync_copy(ee_v.at[0], den_sp.at[eib.at[1]], add=True)
            return 0

        lax.fori_loop(0, _NCH_DEN, den_chunk, 0)
        plsc.subcore_barrier()
        pltpu.sync_copy(den_sp, den_v)

        base_row = et * _E + tile * _E_ROW

        def row_chunk(j, _):
            off = base_row + j * _CH
            pltpu.sync_copy(src_hbm.at[pl.ds(off, _CH)], eib.at[0])
            pltpu.sync_copy(dst_hbm.at[pl.ds(off, _CH)], eib.at[1])
            cp = pltpu.make_async_copy(xs_hbm.at[et].at[eib.at[0]],
                                       rows_v, sem)
            cp.start()
            for q in range(_CH // 16):
                si = eib[0, pl.ds(q * 16, 16)]
                di = eib[1, pl.ds(q * 16, 16)]
                av = plsc.load_gather(as_v, [si])
                bv = plsc.load_gather(ad_v, [di])
                dv = plsc.load_gather(den_v, [di])
                ee_v[0, pl.ds(q * 16, 16)] = (
                    _leaky_exp(av + bv) / (dv + 1e-16))
            cp.wait()

            def scale_edge(e, _):
                a = plsc.load_gather(
                    ee_v, [zi16, jnp.full((16,), e, jnp.int32)])
                for q in range(_D // 16):
                    rows_v[e, pl.ds(q * 16, 16)] = (
                        rows_v[e, pl.ds(q * 16, 16)] * a)
                return 0

            lax.fori_loop(0, _CH, scale_edge, 0)
            pltpu.sync_copy(rows_v, out_sp.at[eib.at[1]], add=True)
            return 0

        lax.fori_loop(0, _NCH_ROW, row_chunk, 0)
        plsc.subcore_barrier()

        pltpu.sync_copy(out_sp.at[pl.ds(s * _ROWS_PER_SUB, _ROWS_PER_SUB)],
                        raw_hbm.at[c, et, pl.ds(s * _ROWS_PER_SUB,
                                                _ROWS_PER_SUB)])
        plsc.subcore_barrier()
        return 0

    lax.fori_loop(0, _ET, per_et, 0)


def _sc_edge_pass(src_flat, dst_flat, a_s, a_d, xs_all):
    mesh = plsc.VectorSubcoreMesh(core_axis_name="c", subcore_axis_name="s")
    kcall = pl.kernel(
        _sc_body,
        mesh=mesh,
        out_type=jax.ShapeDtypeStruct((_NC, _ET, _NP, _D), jnp.float32),
        scratch_types=[
            pltpu.VMEM((_NP,), jnp.float32),
            pltpu.VMEM((_NP,), jnp.float32),
            pltpu.VMEM((_NP,), jnp.float32),
            pltpu.VMEM((2, _CH), jnp.int32),
            pltpu.VMEM((1, _CH), jnp.float32),
            pltpu.VMEM((_CH, _D), jnp.float32),
            pltpu.VMEM_SHARED((_NP,), jnp.float32),
            pltpu.VMEM_SHARED((_NP, _D), jnp.float32),
            pltpu.SemaphoreType.DMA,
        ],
        compiler_params=pltpu.CompilerParams(needs_layout_passes=False),
    )
    return kcall(src_flat, dst_flat, a_s, a_d, xs_all)


def _combine_body(r0a, r0b, r1a, r1b, r2a, r2b, b0, b1, b2, out):
    acc = (r0a[0, 0] + r0b[0, 0] + r1a[0, 0] + r1b[0, 0] + r2a[0, 0]
           + r2b[0, 0] + b0[0, 0] + b1[0, 0] + b2[0, 0])
    out[0] = jnp.maximum(acc, 0.0)


def _combine_relu(raw, bias_l):
    specs = []
    for k in range(3):
        for cc in range(_NC):
            specs.append(pl.BlockSpec(
                (1, 1, _RB, _D),
                lambda nt, i, k=k, cc=cc: (cc, _et_of(nt, k), i, 0)))
    for k in range(3):
        specs.append(pl.BlockSpec((1, 1, _D),
                                  lambda nt, i, k=k: (_et_of(nt, k), 0, 0)))
    return pl.pallas_call(
        _combine_body,
        grid=(_NT, _NB),
        in_specs=specs,
        out_specs=pl.BlockSpec((1, _RB, _D), lambda nt, i: (nt, i, 0)),
        out_shape=jax.ShapeDtypeStruct((_NT, _NP, _D), jnp.float32),
    )(raw, raw, raw, raw, raw, raw,
      bias_l[:, None, :], bias_l[:, None, :], bias_l[:, None, :])


def _final_body(r0a, r0b, r1a, r1b, r2a, r2b, b0, b1, b2, wl, bl, out):
    acc = (r0a[0, 0] + r0b[0, 0] + r1a[0, 0] + r1b[0, 0] + r2a[0, 0]
           + r2b[0, 0] + b0[0, 0] + b1[0, 0] + b2[0, 0])
    h = jnp.maximum(acc, 0.0)
    out[...] = (jnp.dot(h, wl[...], preferred_element_type=jnp.float32)
                + bl[0])


def _final_state(raw, bias_l, W_lin, b_lin):
    ets = _ETS_FOR_DST[0]
    specs = []
    for k in range(3):
        for cc in range(_NC):
            specs.append(pl.BlockSpec(
                (1, 1, _RB, _D),
                lambda i, k=k, cc=cc: (cc, ets[k], i, 0)))
    for k in range(3):
        specs.append(pl.BlockSpec((1, 1, _D), lambda i, k=k: (ets[k], 0, 0)))
    specs.append(pl.BlockSpec((_D, _D), lambda i: (0, 0)))
    specs.append(pl.BlockSpec((1, _D), lambda i: (0, 0)))
    return pl.pallas_call(
        _final_body,
        grid=(_NB,),
        in_specs=specs,
        out_specs=pl.BlockSpec((_RB, _D), lambda i: (i, 0)),
        out_shape=jax.ShapeDtypeStruct((_NP, _D), jnp.float32),
    )(raw, raw, raw, raw, raw, raw,
      bias_l[:, None, :], bias_l[:, None, :], bias_l[:, None, :],
      W_lin, b_lin[None, :])


@jax.jit
def kernel(x, edge_index, W_src, W_dst, att_src, att_dst, bias, W_lin, b_lin):
    xp = jnp.pad(x, ((0, 0), (0, _NP - _N), (0, 0)))
    src_flat = edge_index[:, 0, :].reshape(-1)
    dst_flat = edge_index[:, 1, :].reshape(-1)
    for l in range(_L):
        xs_all, a_s, a_d = _projections(
            xp, W_src[l], W_dst[l], att_src[l], att_dst[l])
        raw = _sc_edge_pass(src_flat, dst_flat, a_s.reshape(-1),
                            a_d.reshape(-1), xs_all)
        if l + 1 < _L:
            xp = _combine_relu(raw, bias[l])
        else:
            out = _final_state(raw, bias[l], W_lin, b_lin)
    return out[:_N]


# speedup vs baseline: 20.3719x; 2.4098x over previous
import functools

import jax
import jax.numpy as jnp
from jax import lax
from jax.experimental import pallas as pl
from jax.experimental.pallas import tpu as pltpu
from jax.experimental.pallas import tpu_sc as plsc

_N = 10000
_NP = 10240
_D = 128
_E = 320000
_NT = 4
_ET = 12
_L = 2
_SRC_T = (0, 0, 0, 1, 1, 1, 2, 2, 2, 3, 3, 3)
_DST_T = (1, 2, 3, 0, 2, 3, 0, 1, 3, 0, 1, 2)
_ETS_FOR_DST = tuple(tuple(et for et in range(_ET) if _DST_T[et] == nt)
                     for nt in range(_NT))

_NC = 2
_NS = 16
_NW = _NC * _NS

_E_ROW = _E // _NW
_CH = 80
_NCH_ROW = _E_ROW // _CH
_E_DEN = _E // _NS
_NCH_DEN = _E_DEN // _CH

_ROWS_PER_SUB = _NP // _NS


def _leaky_exp(t):
    return jnp.exp(jnp.where(t > 0, t, 0.2 * t))


_RB = 1024
_NB = _NP // _RB


def _proj_body(xsrc, xdst, ws, wd, asrc, adst, xs_out, as_out, ad_out):
    xs = jnp.dot(xsrc[0], ws[0], preferred_element_type=jnp.float32)
    xs_out[0] = xs
    as_out[0] = lax.dot_general(asrc[0], xs, (((1,), (1,)), ((), ())),
                                preferred_element_type=jnp.float32)
    w_eff = lax.dot_general(adst[0], wd[0], (((1,), (1,)), ((), ())),
                            preferred_element_type=jnp.float32)
    ad_out[0] = lax.dot_general(w_eff, xdst[0], (((1,), (1,)), ((), ())),
                                preferred_element_type=jnp.float32)


def _src_of(et):
    return et // 3


def _dst_of(et):
    s = et // 3
    r = et % 3
    return jnp.where(r < s, r, r + 1)


def _et_of(nt, k):
    s = jnp.where(k < nt, k, k + 1)
    r = jnp.where(nt < s, nt, nt - 1)
    return 3 * s + r


def _projections(xp, Ws, Wd, att_s, att_d):
    return pl.pallas_call(
        _proj_body,
        grid=(_ET, _NB),
        in_specs=[
            pl.BlockSpec((1, _RB, _D), lambda et, i: (_src_of(et), i, 0)),
            pl.BlockSpec((1, _RB, _D), lambda et, i: (_dst_of(et), i, 0)),
            pl.BlockSpec((1, _D, _D), lambda et, i: (et, 0, 0)),
            pl.BlockSpec((1, _D, _D), lambda et, i: (et, 0, 0)),
            pl.BlockSpec((1, 1, _D), lambda et, i: (et, 0, 0)),
            pl.BlockSpec((1, 1, _D), lambda et, i: (et, 0, 0)),
        ],
        out_specs=[
            pl.BlockSpec((1, _RB, _D), lambda et, i: (et, i, 0)),
            pl.BlockSpec((1, 1, _RB), lambda et, i: (et, 0, i)),
            pl.BlockSpec((1, 1, _RB), lambda et, i: (et, 0, i)),
        ],
        out_shape=[
            jax.ShapeDtypeStruct((_ET, _NP, _D), jnp.float32),
            jax.ShapeDtypeStruct((_ET, 1, _NP), jnp.float32),
            jax.ShapeDtypeStruct((_ET, 1, _NP), jnp.float32),
        ],
    )(xp, xp, Ws, Wd, att_s[:, None, :], att_d[:, None, :])


def _sc_body(src_hbm, dst_hbm, as_hbm, ad_hbm, xs_hbm, raw_hbm, den_hbm,
             as_v, ad_v, eib, ee_v, rows_v,
             den_sp, out_sp,
             gsem, dsem0, dsem1, rsem0, rsem1, is0, is1, is2, is3):
    c = lax.axis_index("c")
    s = lax.axis_index("s")
    tile = c * _NS + s
    z16 = jnp.zeros((16,), jnp.float32)
    zi16 = jnp.zeros((16,), jnp.int32)
    dsem = (dsem0, dsem1)
    rsem = (rsem0, rsem1)
    isem = (is0, is1, is2, is3)
    _LAST = _NCH_ROW - 1

    def per_et(et, _):
        def zrow_init(i, _):
            for q in range(_D // 16):
                rows_v[0, i, pl.ds(q * 16, 16)] = z16
            return 0
        lax.fori_loop(0, _CH, zrow_init, 0)

        def zden_init(i, _):
            as_v[pl.ds(i * 16, 16)] = z16
            return 0
        lax.fori_loop(0, _ROWS_PER_SUB // 16, zden_init, 0)

        for r in range(_ROWS_PER_SUB // _CH):
            pltpu.sync_copy(rows_v.at[0],
                            out_sp.at[pl.ds(s * _ROWS_PER_SUB + r * _CH, _CH)])
        pltpu.sync_copy(as_v.at[pl.ds(0, _ROWS_PER_SUB)],
                        den_sp.at[pl.ds(s * _ROWS_PER_SUB, _ROWS_PER_SUB)])
        pltpu.sync_copy(as_hbm.at[pl.ds(et * _NP, _NP)], as_v)
        pltpu.sync_copy(ad_hbm.at[pl.ds(et * _NP, _NP)], ad_v)
        plsc.subcore_barrier()

        base_row = et * _E + tile * _E_ROW

        def idx_start(jj, r):
            off = base_row + jj * _CH
            pltpu.async_copy(src_hbm.at[pl.ds(off, _CH)], eib.at[r, 0],
                             isem[r])
            pltpu.async_copy(dst_hbm.at[pl.ds(off, _CH)], eib.at[r, 1],
                             isem[r])

        def idx_wait(r):
            pltpu.make_async_copy(src_hbm.at[pl.ds(0, _CH)], eib.at[r, 0],
                                  isem[r]).wait()
            pltpu.make_async_copy(src_hbm.at[pl.ds(0, _CH)], eib.at[r, 1],
                                  isem[r]).wait()

        def gather_start(r, b):
            pltpu.make_async_copy(xs_hbm.at[et].at[eib.at[r, 0]],
                                  rows_v.at[b], gsem).start()

        def gather_wait(r, b):
            pltpu.make_async_copy(xs_hbm.at[et].at[eib.at[r, 0]],
                                  rows_v.at[b], gsem).wait()

        def compute_ee(r, b):
            for q in range(_CH // 16):
                si = eib[r, 0, pl.ds(q * 16, 16)]
                di = eib[r, 1, pl.ds(q * 16, 16)]
                av = plsc.load_gather(as_v, [si])
                bv = plsc.load_gather(ad_v, [di])
                ee_v[b, 0, pl.ds(q * 16, 16)] = _leaky_exp(av + bv)

        def den_scatter_start(r, b):
            pltpu.async_copy(ee_v.at[b, 0], den_sp.at[eib.at[r, 1]],
                             dsem[b], add=True)

        def den_scatter_wait(r, b):
            pltpu.make_async_copy(ee_v.at[b, 0], den_sp.at[eib.at[r, 1]],
                                  dsem[b]).wait()

        def scale(b):
            bsplat = jnp.full((16,), b, jnp.int32)

            def scale_edge(e, _):
                a = plsc.load_gather(
                    ee_v, [bsplat, zi16, jnp.full((16,), e, jnp.int32)])
                for q in range(_D // 16):
                    rows_v[b, e, pl.ds(q * 16, 16)] = (
                        rows_v[b, e, pl.ds(q * 16, 16)] * a)
                return 0

            lax.fori_loop(0, _CH, scale_edge, 0)

        def rows_scatter_start(r, b):
            pltpu.async_copy(rows_v.at[b], out_sp.at[eib.at[r, 1]],
                             rsem[b], add=True)

        def rows_scatter_wait(r, b):
            pltpu.make_async_copy(rows_v.at[b], out_sp.at[eib.at[r, 1]],
                                  rsem[b]).wait()

        idx_start(0, 0)
        idx_wait(0)
        gather_start(0, 0)
        idx_start(1, 1)

        def quad(i, _):
            for u in range(4):
                b = u % 2
                nb = (u + 1) % 2
                nr = (u + 1) % 4
                compute_ee(u, b)
                den_scatter_start(u, b)
                gather_wait(u, b)
                scale(b)
                rows_scatter_start(u, b)

                def drain_prev():
                    den_scatter_wait((u + 3) % 4, nb)
                    rows_scatter_wait((u + 3) % 4, nb)

                if u == 0:
                    @pl.when(i > 0)
                    def _():
                        drain_prev()
                else:
                    drain_prev()

                idx_wait(nr)
                gather_start(nr, nb)
                if u == 3:
                    @pl.when(i < _NCH_ROW // 4 - 1)
                    def _():
                        idx_start(4 * i + u + 2, (u + 2) % 4)
                else:
                    idx_start(4 * i + u + 2, (u + 2) % 4)
            return 0

        lax.fori_loop(0, _NCH_ROW // 4, quad, 0)

        compute_ee(0, 0)
        den_scatter_start(0, 0)
        gather_wait(0, 0)
        scale(0)
        rows_scatter_start(0, 0)
        den_scatter_wait(3, 1)
        rows_scatter_wait(3, 1)
        den_scatter_wait(0, 0)
        rows_scatter_wait(0, 0)
        plsc.subcore_barrier()

        pltpu.sync_copy(out_sp.at[pl.ds(s * _ROWS_PER_SUB, _ROWS_PER_SUB)],
                        raw_hbm.at[c, et, pl.ds(s * _ROWS_PER_SUB,
                                                _ROWS_PER_SUB)])
        pltpu.sync_copy(
            den_sp.at[pl.ds(s * _ROWS_PER_SUB, _ROWS_PER_SUB)],
            den_hbm.at[pl.ds((c * _ET + et) * _NP + s * _ROWS_PER_SUB,
                             _ROWS_PER_SUB)])
        plsc.subcore_barrier()
        return 0

    lax.fori_loop(0, _ET, per_et, 0)


def _sc_edge_pass(src_flat, dst_flat, a_s, a_d, xs_all):
    mesh = plsc.VectorSubcoreMesh(core_axis_name="c", subcore_axis_name="s")
    kcall = pl.kernel(
        _sc_body,
        mesh=mesh,
        out_type=[
            jax.ShapeDtypeStruct((_NC, _ET, _NP, _D), jnp.float32),
            jax.ShapeDtypeStruct((_NC * _ET * _NP,), jnp.float32),
        ],
        scratch_types=[
            pltpu.VMEM((_NP,), jnp.float32),
            pltpu.VMEM((_NP,), jnp.float32),
            pltpu.VMEM((4, 2, _CH), jnp.int32),
            pltpu.VMEM((2, 1, _CH), jnp.float32),
            pltpu.VMEM((2, _CH, _D), jnp.float32),
            pltpu.VMEM_SHARED((_NP,), jnp.float32),
            pltpu.VMEM_SHARED((_NP, _D), jnp.float32),
        ] + [pltpu.SemaphoreType.DMA] * 9,
        compiler_params=pltpu.CompilerParams(needs_layout_passes=False),
    )
    return kcall(src_flat, dst_flat, a_s, a_d, xs_all)


def _acc_of(args):
    (r0a, r0b, r1a, r1b, r2a, r2b,
     d0a, d0b, d1a, d1b, d2a, d2b, b0, b1, b2) = args
    acc = (r0a[0, 0] + r0b[0, 0]) / (d0a[0, 0] + d0b[0, 0] + 1e-16)
    acc += (r1a[0, 0] + r1b[0, 0]) / (d1a[0, 0] + d1b[0, 0] + 1e-16)
    acc += (r2a[0, 0] + r2b[0, 0]) / (d2a[0, 0] + d2b[0, 0] + 1e-16)
    return acc + b0[0, 0] + b1[0, 0] + b2[0, 0]


def _combine_body(*args):
    out = args[-1]
    out[0] = jnp.maximum(_acc_of(args[:-1]), 0.0)


def _rawden_specs(et_fn):
    specs = []
    for k in range(3):
        for cc in range(_NC):
            specs.append(pl.BlockSpec(
                (1, 1, _RB, _D),
                lambda *g, k=k, cc=cc: (cc, et_fn(g, k), g[-1], 0)))
    for k in range(3):
        for cc in range(_NC):
            specs.append(pl.BlockSpec(
                (1, 1, _RB, 1),
                lambda *g, k=k, cc=cc: (cc, et_fn(g, k), g[-1], 0)))
    for k in range(3):
        specs.append(pl.BlockSpec(
            (1, 1, _D), lambda *g, k=k: (et_fn(g, k), 0, 0)))
    return specs


def _combine_relu(raw, den, bias_l):
    specs = _rawden_specs(lambda g, k: _et_of(g[0], k))
    b3 = bias_l[:, None, :]
    return pl.pallas_call(
        _combine_body,
        grid=(_NT, _NB),
        in_specs=specs,
        out_specs=pl.BlockSpec((1, _RB, _D), lambda nt, i: (nt, i, 0)),
        out_shape=jax.ShapeDtypeStruct((_NT, _NP, _D), jnp.float32),
    )(raw, raw, raw, raw, raw, raw, den, den, den, den, den, den, b3, b3, b3)


def _final_body(*args):
    wl, bl, out = args[-3:]
    h = jnp.maximum(_acc_of(args[:-3]), 0.0)
    out[...] = (jnp.dot(h, wl[...], preferred_element_type=jnp.float32)
                + bl[0])


def _final_state(raw, den, bias_l, W_lin, b_lin):
    ets = _ETS_FOR_DST[0]
    specs = _rawden_specs(lambda g, k: ets[k])
    specs.append(pl.BlockSpec((_D, _D), lambda i: (0, 0)))
    specs.append(pl.BlockSpec((1, _D), lambda i: (0, 0)))
    b3 = bias_l[:, None, :]
    return pl.pallas_call(
        _final_body,
        grid=(_NB,),
        in_specs=specs,
        out_specs=pl.BlockSpec((_RB, _D), lambda i: (i, 0)),
        out_shape=jax.ShapeDtypeStruct((_NP, _D), jnp.float32),
    )(raw, raw, raw, raw, raw, raw, den, den, den, den, den, den, b3, b3, b3,
      W_lin, b_lin[None, :])


@jax.jit
def kernel(x, edge_index, W_src, W_dst, att_src, att_dst, bias, W_lin, b_lin):
    xp = jnp.pad(x, ((0, 0), (0, _NP - _N), (0, 0)))
    src_flat = edge_index[:, 0, :].reshape(-1)
    dst_flat = edge_index[:, 1, :].reshape(-1)
    for l in range(_L):
        xs_all, a_s, a_d = _projections(
            xp, W_src[l], W_dst[l], att_src[l], att_dst[l])
        raw, den = _sc_edge_pass(src_flat, dst_flat, a_s.reshape(-1),
                                 a_d.reshape(-1), xs_all)
        den = den.reshape(_NC, _ET, _NP, 1)
        if l + 1 < _L:
            xp = _combine_relu(raw, den, bias[l])
        else:
            out = _final_state(raw, den, bias[l], W_lin, b_lin)
    return out[:_N]


# speedup vs baseline: 21.2873x; 1.0449x over previous
import functools

import jax
import jax.numpy as jnp
from jax import lax
from jax.experimental import pallas as pl
from jax.experimental.pallas import tpu as pltpu
from jax.experimental.pallas import tpu_sc as plsc

_N = 10000
_NP = 10240
_D = 128
_E = 320000
_NT = 4
_ET = 12
_L = 2
_SRC_T = (0, 0, 0, 1, 1, 1, 2, 2, 2, 3, 3, 3)
_DST_T = (1, 2, 3, 0, 2, 3, 0, 1, 3, 0, 1, 2)
_ETS_FOR_DST = tuple(tuple(et for et in range(_ET) if _DST_T[et] == nt)
                     for nt in range(_NT))

_NC = 2
_NS = 16
_NW = _NC * _NS

_E_ROW = _E // _NW
_CH = 96
_NCH_ROW = 104
_TCH = _E_ROW - _NCH_ROW * _CH

_ROWS_PER_SUB = _NP // _NS


def _leaky_exp(t):
    return jnp.exp(jnp.where(t > 0, t, 0.2 * t))


_RB = 1024
_NB = _NP // _RB


def _proj_body(xsrc, xdst, ws, wd, asrc, adst, xs_out, as_out, ad_out):
    xs = jnp.dot(xsrc[0], ws[0], preferred_element_type=jnp.float32)
    xs_out[0] = xs
    as_out[0] = lax.dot_general(asrc[0], xs, (((1,), (1,)), ((), ())),
                                preferred_element_type=jnp.float32)
    w_eff = lax.dot_general(adst[0], wd[0], (((1,), (1,)), ((), ())),
                            preferred_element_type=jnp.float32)
    ad_out[0] = lax.dot_general(w_eff, xdst[0], (((1,), (1,)), ((), ())),
                                preferred_element_type=jnp.float32)


def _src_of(et):
    return et // 3


def _dst_of(et):
    s = et // 3
    r = et % 3
    return jnp.where(r < s, r, r + 1)


def _et_of(nt, k):
    s = jnp.where(k < nt, k, k + 1)
    r = jnp.where(nt < s, nt, nt - 1)
    return 3 * s + r


def _projections(xp, Ws, Wd, att_s, att_d):
    return pl.pallas_call(
        _proj_body,
        grid=(_ET, _NB),
        in_specs=[
            pl.BlockSpec((1, _RB, _D), lambda et, i: (_src_of(et), i, 0)),
            pl.BlockSpec((1, _RB, _D), lambda et, i: (_dst_of(et), i, 0)),
            pl.BlockSpec((1, _D, _D), lambda et, i: (et, 0, 0)),
            pl.BlockSpec((1, _D, _D), lambda et, i: (et, 0, 0)),
            pl.BlockSpec((1, 1, _D), lambda et, i: (et, 0, 0)),
            pl.BlockSpec((1, 1, _D), lambda et, i: (et, 0, 0)),
        ],
        out_specs=[
            pl.BlockSpec((1, _RB, _D), lambda et, i: (et, i, 0)),
            pl.BlockSpec((1, 1, _RB), lambda et, i: (et, 0, i)),
            pl.BlockSpec((1, 1, _RB), lambda et, i: (et, 0, i)),
        ],
        out_shape=[
            jax.ShapeDtypeStruct((_ET, _NP, _D), jnp.float32),
            jax.ShapeDtypeStruct((_ET, 1, _NP), jnp.float32),
            jax.ShapeDtypeStruct((_ET, 1, _NP), jnp.float32),
        ],
    )(xp, xp, Ws, Wd, att_s[:, None, :], att_d[:, None, :])


def _sc_body(src_hbm, dst_hbm, as_hbm, ad_hbm, xs_hbm, raw_hbm, den_hbm,
             as_v, ad_v, eib, ee_v, rows_v, teib, tee,
             den_sp, out_sp,
             gsem, dsem0, dsem1, rsem0, rsem1, is0, is1, is2, is3):
    c = lax.axis_index("c")
    s = lax.axis_index("s")
    tile = c * _NS + s
    z16 = jnp.zeros((16,), jnp.float32)
    zi16 = jnp.zeros((16,), jnp.int32)
    dsem = (dsem0, dsem1)
    rsem = (rsem0, rsem1)
    isem = (is0, is1, is2, is3)

    def per_et(et, _):
        def zrow_init(i, _):
            for q in range(_D // 16):
                rows_v[0, i, pl.ds(q * 16, 16)] = z16
            return 0
        lax.fori_loop(0, _CH, zrow_init, 0)

        def zden_init(i, _):
            as_v[pl.ds(i * 16, 16)] = z16
            return 0
        lax.fori_loop(0, _ROWS_PER_SUB // 16, zden_init, 0)

        for r in range(_ROWS_PER_SUB // _CH):
            pltpu.sync_copy(rows_v.at[0],
                            out_sp.at[pl.ds(s * _ROWS_PER_SUB + r * _CH, _CH)])
        _REM = _ROWS_PER_SUB - (_ROWS_PER_SUB // _CH) * _CH
        if _REM:
            pltpu.sync_copy(
                rows_v.at[0, pl.ds(0, _REM)],
                out_sp.at[pl.ds(s * _ROWS_PER_SUB + _ROWS_PER_SUB - _REM,
                                _REM)])
        pltpu.sync_copy(as_v.at[pl.ds(0, _ROWS_PER_SUB)],
                        den_sp.at[pl.ds(s * _ROWS_PER_SUB, _ROWS_PER_SUB)])
        pltpu.sync_copy(as_hbm.at[pl.ds(et * _NP, _NP)], as_v)
        pltpu.sync_copy(ad_hbm.at[pl.ds(et * _NP, _NP)], ad_v)
        plsc.subcore_barrier()

        base_row = et * _E + tile * _E_ROW

        def idx_start(jj, r):
            off = base_row + jj * _CH
            pltpu.async_copy(src_hbm.at[pl.ds(off, _CH)], eib.at[r, 0],
                             isem[r])
            pltpu.async_copy(dst_hbm.at[pl.ds(off, _CH)], eib.at[r, 1],
                             isem[r])

        def idx_wait(r):
            pltpu.make_async_copy(src_hbm.at[pl.ds(0, _CH)], eib.at[r, 0],
                                  isem[r]).wait()
            pltpu.make_async_copy(src_hbm.at[pl.ds(0, _CH)], eib.at[r, 1],
                                  isem[r]).wait()

        def gather_start(r, b):
            pltpu.make_async_copy(xs_hbm.at[et].at[eib.at[r, 0]],
                                  rows_v.at[b], gsem).start()

        def gather_wait(r, b):
            pltpu.make_async_copy(xs_hbm.at[et].at[eib.at[r, 0]],
                                  rows_v.at[b], gsem).wait()

        def compute_ee(r, b):
            for q in range(_CH // 16):
                si = eib[r, 0, pl.ds(q * 16, 16)]
                di = eib[r, 1, pl.ds(q * 16, 16)]
                av = plsc.load_gather(as_v, [si])
                bv = plsc.load_gather(ad_v, [di])
                ee_v[b, 0, pl.ds(q * 16, 16)] = _leaky_exp(av + bv)

        def den_scatter_start(r, b):
            pltpu.async_copy(ee_v.at[b, 0], den_sp.at[eib.at[r, 1]],
                             dsem[b], add=True)

        def den_scatter_wait(r, b):
            pltpu.make_async_copy(ee_v.at[b, 0], den_sp.at[eib.at[r, 1]],
                                  dsem[b]).wait()

        def scale(b):
            bsplat = jnp.full((16,), b, jnp.int32)

            def scale_edge(e, _):
                a = plsc.load_gather(
                    ee_v, [bsplat, zi16, jnp.full((16,), e, jnp.int32)])
                for q in range(_D // 16):
                    rows_v[b, e, pl.ds(q * 16, 16)] = (
                        rows_v[b, e, pl.ds(q * 16, 16)] * a)
                return 0

            lax.fori_loop(0, _CH, scale_edge, 0)

        def rows_scatter_start(r, b):
            pltpu.async_copy(rows_v.at[b], out_sp.at[eib.at[r, 1]],
                             rsem[b], add=True)

        def rows_scatter_wait(r, b):
            pltpu.make_async_copy(rows_v.at[b], out_sp.at[eib.at[r, 1]],
                                  rsem[b]).wait()

        idx_start(0, 0)
        idx_wait(0)
        gather_start(0, 0)
        idx_start(1, 1)

        def quad(i, _):
            for u in range(4):
                b = u % 2
                nb = (u + 1) % 2
                nr = (u + 1) % 4
                compute_ee(u, b)
                den_scatter_start(u, b)
                gather_wait(u, b)
                scale(b)
                rows_scatter_start(u, b)

                def drain_prev():
                    den_scatter_wait((u + 3) % 4, nb)
                    rows_scatter_wait((u + 3) % 4, nb)

                if u == 0:
                    @pl.when(i > 0)
                    def _():
                        drain_prev()
                else:
                    drain_prev()

                if u == 3:
                    @pl.when(i < _NCH_ROW // 4 - 1)
                    def _():
                        idx_wait(nr)
                        gather_start(nr, nb)
                        idx_start(4 * i + u + 2, (u + 2) % 4)
                else:
                    idx_wait(nr)
                    gather_start(nr, nb)
                    if u == 2:
                        @pl.when(i < _NCH_ROW // 4 - 1)
                        def _():
                            idx_start(4 * i + u + 2, (u + 2) % 4)
                    else:
                        idx_start(4 * i + u + 2, (u + 2) % 4)
            return 0

        lax.fori_loop(0, _NCH_ROW // 4, quad, 0)

        toff = base_row + _NCH_ROW * _CH
        pltpu.sync_copy(src_hbm.at[pl.ds(toff, _TCH)], teib.at[0])
        pltpu.sync_copy(dst_hbm.at[pl.ds(toff, _TCH)], teib.at[1])
        trows = rows_v.at[0, pl.ds(0, _TCH)]
        pltpu.make_async_copy(xs_hbm.at[et].at[teib.at[0]], trows,
                              gsem).start()
        tsi = teib[0, pl.ds(0, 16)]
        tdi = teib[1, pl.ds(0, 16)]
        tav = plsc.load_gather(as_v, [tsi])
        tbv = plsc.load_gather(ad_v, [tdi])
        tee[0, pl.ds(0, 16)] = _leaky_exp(tav + tbv)
        pltpu.async_copy(tee.at[0], den_sp.at[teib.at[1]], dsem0, add=True)
        pltpu.make_async_copy(xs_hbm.at[et].at[teib.at[0]], trows,
                              gsem).wait()

        def tscale_edge(e, _):
            a = plsc.load_gather(tee, [zi16, jnp.full((16,), e, jnp.int32)])
            for q in range(_D // 16):
                rows_v[0, e, pl.ds(q * 16, 16)] = (
                    rows_v[0, e, pl.ds(q * 16, 16)] * a)
            return 0

        lax.fori_loop(0, _TCH, tscale_edge, 0)
        pltpu.async_copy(trows, out_sp.at[teib.at[1]], rsem0, add=True)
        den_scatter_wait(3, 1)
        rows_scatter_wait(3, 1)
        pltpu.make_async_copy(tee.at[0], den_sp.at[teib.at[1]], dsem0).wait()
        pltpu.make_async_copy(trows, out_sp.at[teib.at[1]], rsem0).wait()
        plsc.subcore_barrier()

        pltpu.sync_copy(out_sp.at[pl.ds(s * _ROWS_PER_SUB, _ROWS_PER_SUB)],
                        raw_hbm.at[c, et, pl.ds(s * _ROWS_PER_SUB,
                                                _ROWS_PER_SUB)])
        pltpu.sync_copy(
            den_sp.at[pl.ds(s * _ROWS_PER_SUB, _ROWS_PER_SUB)],
            den_hbm.at[pl.ds((c * _ET + et) * _NP + s * _ROWS_PER_SUB,
                             _ROWS_PER_SUB)])
        plsc.subcore_barrier()
        return 0

    lax.fori_loop(0, _ET, per_et, 0)


def _sc_edge_pass(src_flat, dst_flat, a_s, a_d, xs_all):
    mesh = plsc.VectorSubcoreMesh(core_axis_name="c", subcore_axis_name="s")
    kcall = pl.kernel(
        _sc_body,
        mesh=mesh,
        out_type=[
            jax.ShapeDtypeStruct((_NC, _ET, _NP, _D), jnp.float32),
            jax.ShapeDtypeStruct((_NC * _ET * _NP,), jnp.float32),
        ],
        scratch_types=[
            pltpu.VMEM((_NP,), jnp.float32),
            pltpu.VMEM((_NP,), jnp.float32),
            pltpu.VMEM((4, 2, _CH), jnp.int32),
            pltpu.VMEM((2, 1, _CH), jnp.float32),
            pltpu.VMEM((2, _CH, _D), jnp.float32),
            pltpu.VMEM((2, _TCH), jnp.int32),
            pltpu.VMEM((1, _TCH), jnp.float32),
            pltpu.VMEM_SHARED((_NP,), jnp.float32),
            pltpu.VMEM_SHARED((_NP, _D), jnp.float32),
        ] + [pltpu.SemaphoreType.DMA] * 9,
        compiler_params=pltpu.CompilerParams(needs_layout_passes=False),
    )
    return kcall(src_flat, dst_flat, a_s, a_d, xs_all)


def _acc_of(args):
    (r0a, r0b, r1a, r1b, r2a, r2b,
     d0a, d0b, d1a, d1b, d2a, d2b, b0, b1, b2) = args
    acc = (r0a[0, 0] + r0b[0, 0]) / (d0a[0, 0] + d0b[0, 0] + 1e-16)
    acc += (r1a[0, 0] + r1b[0, 0]) / (d1a[0, 0] + d1b[0, 0] + 1e-16)
    acc += (r2a[0, 0] + r2b[0, 0]) / (d2a[0, 0] + d2b[0, 0] + 1e-16)
    return acc + b0[0, 0] + b1[0, 0] + b2[0, 0]


def _combine_body(*args):
    out = args[-1]
    out[0] = jnp.maximum(_acc_of(args[:-1]), 0.0)


def _rawden_specs(et_fn):
    specs = []
    for k in range(3):
        for cc in range(_NC):
            specs.append(pl.BlockSpec(
                (1, 1, _RB, _D),
                lambda *g, k=k, cc=cc: (cc, et_fn(g, k), g[-1], 0)))
    for k in range(3):
        for cc in range(_NC):
            specs.append(pl.BlockSpec(
                (1, 1, _RB, 1),
                lambda *g, k=k, cc=cc: (cc, et_fn(g, k), g[-1], 0)))
    for k in range(3):
        specs.append(pl.BlockSpec(
            (1, 1, _D), lambda *g, k=k: (et_fn(g, k), 0, 0)))
    return specs


def _combine_relu(raw, den, bias_l):
    specs = _rawden_specs(lambda g, k: _et_of(g[0], k))
    b3 = bias_l[:, None, :]
    return pl.pallas_call(
        _combine_body,
        grid=(_NT, _NB),
        in_specs=specs,
        out_specs=pl.BlockSpec((1, _RB, _D), lambda nt, i: (nt, i, 0)),
        out_shape=jax.ShapeDtypeStruct((_NT, _NP, _D), jnp.float32),
    )(raw, raw, raw, raw, raw, raw, den, den, den, den, den, den, b3, b3, b3)


def _final_body(*args):
    wl, bl, out = args[-3:]
    h = jnp.maximum(_acc_of(args[:-3]), 0.0)
    out[...] = (jnp.dot(h, wl[...], preferred_element_type=jnp.float32)
                + bl[0])


def _final_state(raw, den, bias_l, W_lin, b_lin):
    ets = _ETS_FOR_DST[0]
    specs = _rawden_specs(lambda g, k: ets[k])
    specs.append(pl.BlockSpec((_D, _D), lambda i: (0, 0)))
    specs.append(pl.BlockSpec((1, _D), lambda i: (0, 0)))
    b3 = bias_l[:, None, :]
    return pl.pallas_call(
        _final_body,
        grid=(_NB,),
        in_specs=specs,
        out_specs=pl.BlockSpec((_RB, _D), lambda i: (i, 0)),
        out_shape=jax.ShapeDtypeStruct((_NP, _D), jnp.float32),
    )(raw, raw, raw, raw, raw, raw, den, den, den, den, den, den, b3, b3, b3,
      W_lin, b_lin[None, :])


@jax.jit
def kernel(x, edge_index, W_src, W_dst, att_src, att_dst, bias, W_lin, b_lin):
    xp = jnp.pad(x, ((0, 0), (0, _NP - _N), (0, 0)))
    src_flat = edge_index[:, 0, :].reshape(-1)
    dst_flat = edge_index[:, 1, :].reshape(-1)
    for l in range(_L):
        xs_all, a_s, a_d = _projections(
            xp, W_src[l], W_dst[l], att_src[l], att_dst[l])
        raw, den = _sc_edge_pass(src_flat, dst_flat, a_s.reshape(-1),
                                 a_d.reshape(-1), xs_all)
        den = den.reshape(_NC, _ET, _NP, 1)
        if l + 1 < _L:
            xp = _combine_relu(raw, den, bias[l])
        else:
            out = _final_state(raw, den, bias[l], W_lin, b_lin)
    return out[:_N]


# speedup vs baseline: 22.0440x; 1.0355x over previous
import functools

import jax
import jax.numpy as jnp
from jax import lax
from jax.experimental import pallas as pl
from jax.experimental.pallas import tpu as pltpu
from jax.experimental.pallas import tpu_sc as plsc

_N = 10000
_NP = 10240
_D = 128
_E = 320000
_NT = 4
_ET = 12
_L = 2
_SRC_T = (0, 0, 0, 1, 1, 1, 2, 2, 2, 3, 3, 3)
_DST_T = (1, 2, 3, 0, 2, 3, 0, 1, 3, 0, 1, 2)
_ETS_FOR_DST = tuple(tuple(et for et in range(_ET) if _DST_T[et] == nt)
                     for nt in range(_NT))

_NC = 2
_NS = 16
_NW = _NC * _NS

_E_ROW = _E // _NW
_CH = 96
_NCH_ROW = 104
_TCH = _E_ROW - _NCH_ROW * _CH

_ROWS_PER_SUB = _NP // _NS


def _leaky_exp(t):
    return jnp.exp(jnp.where(t > 0, t, 0.2 * t))


_RB = 1024
_NB = _NP // _RB


def _proj_body(xsrc, xdst, ws, wd, asrc, adst, xs_out, as_out, ad_out):
    xs = jnp.dot(xsrc[0], ws[0], preferred_element_type=jnp.float32)
    xs_out[0] = xs
    as_out[0] = lax.dot_general(asrc[0], xs, (((1,), (1,)), ((), ())),
                                preferred_element_type=jnp.float32)
    w_eff = lax.dot_general(adst[0], wd[0], (((1,), (1,)), ((), ())),
                            preferred_element_type=jnp.float32)
    ad_out[0] = lax.dot_general(w_eff, xdst[0], (((1,), (1,)), ((), ())),
                                preferred_element_type=jnp.float32)


def _src_of(et):
    return et // 3


def _dst_of(et):
    s = et // 3
    r = et % 3
    return jnp.where(r < s, r, r + 1)


def _et_of(nt, k):
    s = jnp.where(k < nt, k, k + 1)
    r = jnp.where(nt < s, nt, nt - 1)
    return 3 * s + r


def _projections(xp, Ws, Wd, att_s, att_d):
    return pl.pallas_call(
        _proj_body,
        grid=(_ET, _NB),
        in_specs=[
            pl.BlockSpec((1, _RB, _D), lambda et, i: (_src_of(et), i, 0)),
            pl.BlockSpec((1, _RB, _D), lambda et, i: (_dst_of(et), i, 0)),
            pl.BlockSpec((1, _D, _D), lambda et, i: (et, 0, 0)),
            pl.BlockSpec((1, _D, _D), lambda et, i: (et, 0, 0)),
            pl.BlockSpec((1, 1, _D), lambda et, i: (et, 0, 0)),
            pl.BlockSpec((1, 1, _D), lambda et, i: (et, 0, 0)),
        ],
        out_specs=[
            pl.BlockSpec((1, _RB, _D), lambda et, i: (et, i, 0)),
            pl.BlockSpec((1, 1, _RB), lambda et, i: (et, 0, i)),
            pl.BlockSpec((1, 1, _RB), lambda et, i: (et, 0, i)),
        ],
        out_shape=[
            jax.ShapeDtypeStruct((_ET, _NP, _D), jnp.float32),
            jax.ShapeDtypeStruct((_ET, 1, _NP), jnp.float32),
            jax.ShapeDtypeStruct((_ET, 1, _NP), jnp.float32),
        ],
    )(xp, xp, Ws, Wd, att_s[:, None, :], att_d[:, None, :])


def _sc_body(src_hbm, dst_hbm, as_hbm, ad_hbm, xs_hbm, raw_hbm, den_hbm,
             as_v, ad_v, eib, ee_v, rows_v, teib, tee,
             den_sp, out_sp,
             gsem, dsem0, dsem1, rsem0, rsem1, is0, is1, is2, is3):
    c = lax.axis_index("c")
    s = lax.axis_index("s")
    tile = c * _NS + s
    z16 = jnp.zeros((16,), jnp.float32)
    zi16 = jnp.zeros((16,), jnp.int32)
    dsem = (dsem0, dsem1)
    rsem = (rsem0, rsem1)
    isem = (is0, is1, is2, is3)

    def per_et(et, _):
        def zrow_init(i, _):
            for q in range(_D // 16):
                rows_v[0, i, pl.ds(q * 16, 16)] = z16
            return 0
        lax.fori_loop(0, _CH, zrow_init, 0)

        def zden_init(i, _):
            as_v[pl.ds(i * 16, 16)] = z16
            return 0
        lax.fori_loop(0, _ROWS_PER_SUB // 16, zden_init, 0)

        for r in range(_ROWS_PER_SUB // _CH):
            pltpu.sync_copy(rows_v.at[0],
                            out_sp.at[pl.ds(s * _ROWS_PER_SUB + r * _CH, _CH)])
        _REM = _ROWS_PER_SUB - (_ROWS_PER_SUB // _CH) * _CH
        if _REM:
            pltpu.sync_copy(
                rows_v.at[0, pl.ds(0, _REM)],
                out_sp.at[pl.ds(s * _ROWS_PER_SUB + _ROWS_PER_SUB - _REM,
                                _REM)])
        pltpu.sync_copy(as_v.at[pl.ds(0, _ROWS_PER_SUB)],
                        den_sp.at[pl.ds(s * _ROWS_PER_SUB, _ROWS_PER_SUB)])
        pltpu.sync_copy(as_hbm.at[pl.ds(et * _NP, _NP)], as_v)
        pltpu.sync_copy(ad_hbm.at[pl.ds(et * _NP, _NP)], ad_v)
        plsc.subcore_barrier()

        base_row = et * _E + tile * _E_ROW

        def idx_start(jj, r):
            off = base_row + jj * _CH
            pltpu.async_copy(src_hbm.at[pl.ds(off, _CH)], eib.at[r, 0],
                             isem[r])
            pltpu.async_copy(dst_hbm.at[pl.ds(off, _CH)], eib.at[r, 1],
                             isem[r])

        def idx_wait(r):
            pltpu.make_async_copy(src_hbm.at[pl.ds(0, _CH)], eib.at[r, 0],
                                  isem[r]).wait()
            pltpu.make_async_copy(src_hbm.at[pl.ds(0, _CH)], eib.at[r, 1],
                                  isem[r]).wait()

        def gather_start(r, b):
            pltpu.make_async_copy(xs_hbm.at[et].at[eib.at[r, 0]],
                                  rows_v.at[b], gsem).start()

        def gather_wait(r, b):
            pltpu.make_async_copy(xs_hbm.at[et].at[eib.at[r, 0]],
                                  rows_v.at[b], gsem).wait()

        def compute_ee(r, b):
            for q in range(_CH // 16):
                si = eib[r, 0, pl.ds(q * 16, 16)]
                di = eib[r, 1, pl.ds(q * 16, 16)]
                av = plsc.load_gather(as_v, [si])
                bv = plsc.load_gather(ad_v, [di])
                ee_v[b, 0, pl.ds(q * 16, 16)] = _leaky_exp(av + bv)

        def den_scatter_start(r, b):
            pltpu.async_copy(ee_v.at[b, 0], den_sp.at[eib.at[r, 1]],
                             dsem[b], add=True)

        def den_scatter_wait(r, b):
            pltpu.make_async_copy(ee_v.at[b, 0], den_sp.at[eib.at[r, 1]],
                                  dsem[b]).wait()

        def scale(b):
            bsplat = jnp.full((16,), b, jnp.int32)

            def scale_quad(i, _):
                for du in range(4):
                    e = 4 * i + du
                    a = plsc.load_gather(
                        ee_v, [bsplat, zi16, jnp.full((16,), e, jnp.int32)])
                    for q in range(_D // 16):
                        rows_v[b, e, pl.ds(q * 16, 16)] = (
                            rows_v[b, e, pl.ds(q * 16, 16)] * a)
                return 0

            lax.fori_loop(0, _CH // 4, scale_quad, 0)

        def rows_scatter_start(r, b):
            pltpu.async_copy(rows_v.at[b], out_sp.at[eib.at[r, 1]],
                             rsem[b], add=True)

        def rows_scatter_wait(r, b):
            pltpu.make_async_copy(rows_v.at[b], out_sp.at[eib.at[r, 1]],
                                  rsem[b]).wait()

        idx_start(0, 0)
        idx_wait(0)
        gather_start(0, 0)
        idx_start(1, 1)

        def quad(i, _):
            for u in range(4):
                b = u % 2
                nb = (u + 1) % 2
                nr = (u + 1) % 4
                compute_ee(u, b)
                den_scatter_start(u, b)
                gather_wait(u, b)
                scale(b)
                rows_scatter_start(u, b)

                def drain_prev():
                    den_scatter_wait((u + 3) % 4, nb)
                    rows_scatter_wait((u + 3) % 4, nb)

                if u == 0:
                    @pl.when(i > 0)
                    def _():
                        drain_prev()
                else:
                    drain_prev()

                if u == 3:
                    @pl.when(i < _NCH_ROW // 4 - 1)
                    def _():
                        idx_wait(nr)
                        gather_start(nr, nb)
                        idx_start(4 * i + u + 2, (u + 2) % 4)
                else:
                    idx_wait(nr)
                    gather_start(nr, nb)
                    if u == 2:
                        @pl.when(i < _NCH_ROW // 4 - 1)
                        def _():
                            idx_start(4 * i + u + 2, (u + 2) % 4)
                    else:
                        idx_start(4 * i + u + 2, (u + 2) % 4)
            return 0

        lax.fori_loop(0, _NCH_ROW // 4, quad, 0)

        toff = base_row + _NCH_ROW * _CH
        pltpu.sync_copy(src_hbm.at[pl.ds(toff, _TCH)], teib.at[0])
        pltpu.sync_copy(dst_hbm.at[pl.ds(toff, _TCH)], teib.at[1])
        trows = rows_v.at[0, pl.ds(0, _TCH)]
        pltpu.make_async_copy(xs_hbm.at[et].at[teib.at[0]], trows,
                              gsem).start()
        tsi = teib[0, pl.ds(0, 16)]
        tdi = teib[1, pl.ds(0, 16)]
        tav = plsc.load_gather(as_v, [tsi])
        tbv = plsc.load_gather(ad_v, [tdi])
        tee[0, pl.ds(0, 16)] = _leaky_exp(tav + tbv)
        pltpu.async_copy(tee.at[0], den_sp.at[teib.at[1]], dsem0, add=True)
        pltpu.make_async_copy(xs_hbm.at[et].at[teib.at[0]], trows,
                              gsem).wait()

        def tscale_edge(e, _):
            a = plsc.load_gather(tee, [zi16, jnp.full((16,), e, jnp.int32)])
            for q in range(_D // 16):
                rows_v[0, e, pl.ds(q * 16, 16)] = (
                    rows_v[0, e, pl.ds(q * 16, 16)] * a)
            return 0

        lax.fori_loop(0, _TCH, tscale_edge, 0)
        pltpu.async_copy(trows, out_sp.at[teib.at[1]], rsem0, add=True)
        den_scatter_wait(3, 1)
        rows_scatter_wait(3, 1)
        pltpu.make_async_copy(tee.at[0], den_sp.at[teib.at[1]], dsem0).wait()
        pltpu.make_async_copy(trows, out_sp.at[teib.at[1]], rsem0).wait()
        plsc.subcore_barrier()

        pltpu.sync_copy(out_sp.at[pl.ds(s * _ROWS_PER_SUB, _ROWS_PER_SUB)],
                        raw_hbm.at[c, et, pl.ds(s * _ROWS_PER_SUB,
                                                _ROWS_PER_SUB)])
        pltpu.sync_copy(
            den_sp.at[pl.ds(s * _ROWS_PER_SUB, _ROWS_PER_SUB)],
            den_hbm.at[pl.ds((c * _ET + et) * _NP + s * _ROWS_PER_SUB,
                             _ROWS_PER_SUB)])
        plsc.subcore_barrier()
        return 0

    lax.fori_loop(0, _ET, per_et, 0)


def _sc_edge_pass(src_flat, dst_flat, a_s, a_d, xs_all):
    mesh = plsc.VectorSubcoreMesh(core_axis_name="c", subcore_axis_name="s")
    kcall = pl.kernel(
        _sc_body,
        mesh=mesh,
        out_type=[
            jax.ShapeDtypeStruct((_NC, _ET, _NP, _D), jnp.float32),
            jax.ShapeDtypeStruct((_NC * _ET * _NP,), jnp.float32),
        ],
        scratch_types=[
            pltpu.VMEM((_NP,), jnp.float32),
            pltpu.VMEM((_NP,), jnp.float32),
            pltpu.VMEM((4, 2, _CH), jnp.int32),
            pltpu.VMEM((2, 1, _CH), jnp.float32),
            pltpu.VMEM((2, _CH, _D), jnp.float32),
            pltpu.VMEM((2, _TCH), jnp.int32),
            pltpu.VMEM((1, _TCH), jnp.float32),
            pltpu.VMEM_SHARED((_NP,), jnp.float32),
            pltpu.VMEM_SHARED((_NP, _D), jnp.float32),
        ] + [pltpu.SemaphoreType.DMA] * 9,
        compiler_params=pltpu.CompilerParams(needs_layout_passes=False),
    )
    return kcall(src_flat, dst_flat, a_s, a_d, xs_all)


def _acc_of(args):
    (r0a, r0b, r1a, r1b, r2a, r2b,
     d0a, d0b, d1a, d1b, d2a, d2b, b0, b1, b2) = args
    acc = (r0a[0, 0] + r0b[0, 0]) / (d0a[0, 0] + d0b[0, 0] + 1e-16)
    acc += (r1a[0, 0] + r1b[0, 0]) / (d1a[0, 0] + d1b[0, 0] + 1e-16)
    acc += (r2a[0, 0] + r2b[0, 0]) / (d2a[0, 0] + d2b[0, 0] + 1e-16)
    return acc + b0[0, 0] + b1[0, 0] + b2[0, 0]


def _combine_body(*args):
    out = args[-1]
    out[0] = jnp.maximum(_acc_of(args[:-1]), 0.0)


def _rawden_specs(et_fn):
    specs = []
    for k in range(3):
        for cc in range(_NC):
            specs.append(pl.BlockSpec(
                (1, 1, _RB, _D),
                lambda *g, k=k, cc=cc: (cc, et_fn(g, k), g[-1], 0)))
    for k in range(3):
        for cc in range(_NC):
            specs.append(pl.BlockSpec(
                (1, 1, _RB, 1),
                lambda *g, k=k, cc=cc: (cc, et_fn(g, k), g[-1], 0)))
    for k in range(3):
        specs.append(pl.BlockSpec(
            (1, 1, _D), lambda *g, k=k: (et_fn(g, k), 0, 0)))
    return specs


def _combine_relu(raw, den, bias_l):
    specs = _rawden_specs(lambda g, k: _et_of(g[0], k))
    b3 = bias_l[:, None, :]
    return pl.pallas_call(
        _combine_body,
        grid=(_NT, _NB),
        in_specs=specs,
        out_specs=pl.BlockSpec((1, _RB, _D), lambda nt, i: (nt, i, 0)),
        out_shape=jax.ShapeDtypeStruct((_NT, _NP, _D), jnp.float32),
    )(raw, raw, raw, raw, raw, raw, den, den, den, den, den, den, b3, b3, b3)


def _final_body(*args):
    wl, bl, out = args[-3:]
    h = jnp.maximum(_acc_of(args[:-3]), 0.0)
    out[...] = (jnp.dot(h, wl[...], preferred_element_type=jnp.float32)
                + bl[0])


def _final_state(raw, den, bias_l, W_lin, b_lin):
    ets = _ETS_FOR_DST[0]
    specs = _rawden_specs(lambda g, k: ets[k])
    specs.append(pl.BlockSpec((_D, _D), lambda i: (0, 0)))
    specs.append(pl.BlockSpec((1, _D), lambda i: (0, 0)))
    b3 = bias_l[:, None, :]
    return pl.pallas_call(
        _final_body,
        grid=(_NB,),
        in_specs=specs,
        out_specs=pl.BlockSpec((_RB, _D), lambda i: (i, 0)),
        out_shape=jax.ShapeDtypeStruct((_NP, _D), jnp.float32),
    )(raw, raw, raw, raw, raw, raw, den, den, den, den, den, den, b3, b3, b3,
      W_lin, b_lin[None, :])


@jax.jit
def kernel(x, edge_index, W_src, W_dst, att_src, att_dst, bias, W_lin, b_lin):
    xp = jnp.pad(x, ((0, 0), (0, _NP - _N), (0, 0)))
    src_flat = edge_index[:, 0, :].reshape(-1)
    dst_flat = edge_index[:, 1, :].reshape(-1)
    for l in range(_L):
        xs_all, a_s, a_d = _projections(
            xp, W_src[l], W_dst[l], att_src[l], att_dst[l])
        raw, den = _sc_edge_pass(src_flat, dst_flat, a_s.reshape(-1),
                                 a_d.reshape(-1), xs_all)
        den = den.reshape(_NC, _ET, _NP, 1)
        if l + 1 < _L:
            xp = _combine_relu(raw, den, bias[l])
        else:
            out = _final_state(raw, den, bias[l], W_lin, b_lin)
    return out[:_N]
